# baseline (speedup 1.0000x reference)
import functools

import jax
import jax.numpy as jnp
from jax import lax
from jax.experimental import pallas as pl
from jax.experimental.pallas import tpu as pltpu
from jax.experimental.pallas import tpu_sc as plsc

_NC = 2
_NS = 16
_NW = _NC * _NS
_DEPTH = 3
_K = 40

_N_TC = 2048
_TC_STEP = 128


@functools.lru_cache(maxsize=None)
def _sc_gather_call(n_sc, n, d):
    per_w = n_sc // _NW
    sizes = [_K] * (per_w // _K)
    if per_w % _K:
        sizes.append(per_w % _K)
    offs = [sum(sizes[:i]) for i in range(len(sizes))]
    nch = len(sizes)
    mesh = plsc.VectorSubcoreMesh(core_axis_name="c", subcore_axis_name="s")

    @functools.partial(
        pl.kernel,
        mesh=mesh,
        out_type=jax.ShapeDtypeStruct((n, d), jnp.float32),
        scratch_types=[pltpu.VMEM((per_w,), jnp.int32)]
        + [pltpu.VMEM((_K, d), jnp.float32)] * _DEPTH
        + [pltpu.SemaphoreType.DMA] * (2 * _DEPTH),
    )
    def grab(ids_hbm, table_hbm, out_hbm, idx_v, *rest):
        bufs = rest[:_DEPTH]
        gsems = rest[_DEPTH:2 * _DEPTH]
        osems = rest[2 * _DEPTH:]
        wid = lax.axis_index("s") * _NC + lax.axis_index("c")
        base = wid * per_w
        pltpu.sync_copy(ids_hbm.at[pl.ds(base, per_w)], idx_v)

        def gather(ch, b):
            sz = sizes[ch]
            dst = bufs[b] if sz == _K else bufs[b].at[pl.ds(0, sz)]
            return pltpu.async_copy(
                table_hbm.at[idx_v.at[pl.ds(offs[ch], sz)]], dst, gsems[b])

        ghandles = [None] * _DEPTH
        ohandles = [None] * _DEPTH
        for ch in range(min(_DEPTH, nch)):
            ghandles[ch] = gather(ch, ch)
        for ch in range(nch):
            b = ch % _DEPTH
            sz = sizes[ch]
            ghandles[b].wait()
            src = bufs[b] if sz == _K else bufs[b].at[pl.ds(0, sz)]
            ohandles[b] = pltpu.async_copy(
                src, out_hbm.at[pl.ds(base + offs[ch], sz)], osems[b])
            nxt = ch + _DEPTH
            if nxt < nch:
                ohandles[b].wait()
                ghandles[b] = gather(nxt, b)
        for h in ohandles:
            if h is not None:
                h.wait()

    return grab


@functools.lru_cache(maxsize=None)
def _tc_gather_call(n_tc, d):
    grid_spec = pltpu.PrefetchScalarGridSpec(
        num_scalar_prefetch=1,
        grid=(n_tc // _TC_STEP,),
        in_specs=[pl.BlockSpec(memory_space=pl.ANY)],
        out_specs=pl.BlockSpec(memory_space=pl.ANY),
        scratch_shapes=[pltpu.SemaphoreType.DMA((_TC_STEP,))],
    )

    def body(ids_ref, table_ref, out_ref, sems):
        g = pl.program_id(0)
        base = g * _TC_STEP
        handles = []
        for j in range(_TC_STEP):
            row = ids_ref[base + j]
            h = pltpu.make_async_copy(
                table_ref.at[row], out_ref.at[base + j], sems.at[j])
            h.start()
            handles.append(h)
        for h in handles:
            h.wait()

    return pl.pallas_call(
        body,
        grid_spec=grid_spec,
        out_shape=jax.ShapeDtypeStruct((n_tc, d), jnp.float32),
    )


def kernel(input_ids, embed_table):
    b, s = input_ids.shape
    v, d = embed_table.shape
    n = b * s
    n_sc = n - _N_TC
    ids = input_ids.reshape(n).astype(jnp.int32)
    out_sc = _sc_gather_call(n_sc, n, d)(ids, embed_table)
    out_tc = _tc_gather_call(_N_TC, d)(ids[n_sc:], embed_table)
    out = lax.dynamic_update_slice(out_sc, out_tc, (n_sc, 0))
    return out.reshape(b, s, d)


# speedup vs baseline: 4.5413x; 4.5413x over previous
import functools

import jax
import jax.numpy as jnp
from jax import lax
from jax.experimental import pallas as pl
from jax.experimental.pallas import tpu as pltpu
from jax.experimental.pallas import tpu_sc as plsc

_NC = 2
_NS = 16
_NW = _NC * _NS
_DEPTH = 3
_K = 40


@functools.lru_cache(maxsize=None)
def _gather_call(n, seq_len, d):
    per_w = n // _NW
    sizes = [_K] * (per_w // _K)
    if per_w % _K:
        sizes.append(per_w % _K)
    offs = [sum(sizes[:i]) for i in range(len(sizes))]
    nch = len(sizes)
    mesh = plsc.VectorSubcoreMesh(core_axis_name="c", subcore_axis_name="s")

    @functools.partial(
        pl.kernel,
        mesh=mesh,
        out_type=jax.ShapeDtypeStruct((n // seq_len, seq_len, d), jnp.float32),
        scratch_types=[pltpu.VMEM((per_w,), jnp.int32)]
        + [pltpu.VMEM((_K, d), jnp.float32)] * _DEPTH
        + [pltpu.SemaphoreType.DMA] * (2 * _DEPTH),
    )
    def grab(ids_hbm, table_hbm, out_hbm, idx_v, *rest):
        bufs = rest[:_DEPTH]
        gsems = rest[_DEPTH:2 * _DEPTH]
        osems = rest[2 * _DEPTH:]
        wid = lax.axis_index("s") * _NC + lax.axis_index("c")
        base = wid * per_w
        seq = ids_hbm.shape[1]
        w_per_row = seq // per_w
        pltpu.sync_copy(
            ids_hbm.at[wid // w_per_row, pl.ds((wid % w_per_row) * per_w, per_w)],
            idx_v)

        def gather(ch, b):
            sz = sizes[ch]
            dst = bufs[b] if sz == _K else bufs[b].at[pl.ds(0, sz)]
            return pltpu.async_copy(
                table_hbm.at[idx_v.at[pl.ds(offs[ch], sz)]], dst, gsems[b])

        ghandles = [None] * _DEPTH
        ohandles = [None] * _DEPTH
        for ch in range(min(_DEPTH, nch)):
            ghandles[ch] = gather(ch, ch)
        for ch in range(nch):
            b = ch % _DEPTH
            sz = sizes[ch]
            ghandles[b].wait()
            src = bufs[b] if sz == _K else bufs[b].at[pl.ds(0, sz)]
            row = (base + offs[ch]) // seq_len
            col = (base + offs[ch]) % seq_len
            ohandles[b] = pltpu.async_copy(
                src, out_hbm.at[row, pl.ds(col, sz)], osems[b])
            nxt = ch + _DEPTH
            if nxt < nch:
                ohandles[b].wait()
                ghandles[b] = gather(nxt, b)
        for h in ohandles:
            if h is not None:
                h.wait()

    return grab


def kernel(input_ids, embed_table):
    b, s = input_ids.shape
    v, d = embed_table.shape
    n = b * s
    return _gather_call(n, s, d)(input_ids, embed_table)
